# baseline (speedup 1.0000x reference)
import functools

import jax
import jax.numpy as jnp
from jax import lax
from jax.experimental import pallas as pl
from jax.experimental.pallas import tpu as pltpu
from jax.experimental.pallas import tpu_sc as plsc

N = 10000
NP = 10240
E = 320000
NC, NS = 2, 16
NW = NC * NS
B = 128
NB = 80
EPW = B * NB
EPAD = EPW * NW
NSLOT = 4
RPT = NP // NS


def _edge_scatter(C, sub):
  mesh = plsc.VectorSubcoreMesh(core_axis_name="c", subcore_axis_name="s")
  bb = B * sub
  nbb = EPW // bb

  scratch = [
      pltpu.VMEM((EPW,), jnp.int32),
      pltpu.VMEM((EPW,), jnp.float32),
      pltpu.VMEM_SHARED((NP, C), jnp.float32),
  ]
  for _ in range(NSLOT):
    scratch.append(pltpu.VMEM((sub, B), jnp.int32))
  for _ in range(NSLOT):
    scratch.append(pltpu.VMEM((sub, B, C), jnp.float32))
  scratch += [pltpu.SemaphoreType.DMA] * (2 * NSLOT)

  @functools.partial(
      pl.kernel,
      out_type=jax.ShapeDtypeStruct((NC, NP, C), jnp.float32),
      mesh=mesh,
      scratch_types=scratch,
      compiler_params=pltpu.CompilerParams(use_tc_tiling_on_sc=False),
  )
  def k(y_hbm, row_hbm, col_hbm, ew_hbm, z_hbm, out_hbm, row_v, ew_v, acc,
        *bufs):
    colb = bufs[0:NSLOT]
    msg = bufs[NSLOT:2 * NSLOT]
    gsem = bufs[2 * NSLOT:3 * NSLOT]
    ssem = bufs[3 * NSLOT:4 * NSLOT]
    cid = lax.axis_index("c")
    sid = lax.axis_index("s")
    wid = sid * NC + cid
    ebase = pl.multiple_of(wid * EPW, 8)
    pltpu.sync_copy(row_hbm.at[pl.ds(ebase, EPW)], row_v)
    pltpu.sync_copy(ew_hbm.at[pl.ds(ebase, EPW)], ew_v)
    rbase = pl.multiple_of(sid * RPT, 8)
    pltpu.sync_copy(z_hbm.at[pl.ds(rbase, RPT)], acc.at[pl.ds(rbase, RPT)])
    plsc.subcore_barrier()

    def gather_start(b, j):
      off = pl.multiple_of(b * bb, 8)
      for si in range(sub):
        pltpu.async_copy(col_hbm.at[pl.ds(ebase + off + si * B, B)],
                         colb[j].at[si], gsem[j])
        pltpu.async_copy(y_hbm.at[row_v.at[pl.ds(off + si * B, B)]],
                         msg[j].at[si], gsem[j])

    def gather_wait(j):
      for si in range(sub):
        pltpu.make_async_copy(col_hbm.at[pl.ds(0, B)], colb[j].at[si],
                              gsem[j]).wait()
        pltpu.make_async_copy(y_hbm.at[row_v.at[pl.ds(0, B)]], msg[j].at[si],
                              gsem[j]).wait()

    def scatter_start(j):
      for si in range(sub):
        pltpu.async_copy(msg[j].at[si], acc.at[colb[j].at[si]], ssem[j],
                         add=True)

    def scatter_wait(j):
      for si in range(sub):
        pltpu.make_async_copy(msg[j].at[si], acc.at[colb[j].at[si]],
                              ssem[j]).wait()

    def scale(b, j):
      off = pl.multiple_of(b * bb, 8)
      for si in range(sub):

        def grp(g, c2):
          ew16 = ew_v[pl.ds(off + si * B + g * 16, 16)]
          for jj in range(16):
            e = g * 16 + jj
            s = ew16.at[jnp.full((16,), jj, jnp.int32)].get(
                mode="promise_in_bounds")
            for cc in range(C // 16):
              msg[j][si, e, pl.ds(cc * 16, 16)] = (
                  msg[j][si, e, pl.ds(cc * 16, 16)] * s)
          return c2

        lax.fori_loop(0, B // 16, grp, 0)

    gather_start(0, 0)
    gather_start(1, 1)

    def outer(i, carry):
      for jj in range(NSLOT):
        b = i * NSLOT + jj
        gather_wait(jj)
        scale(b, jj)
        scatter_start(jj)
        j2 = (jj + 2) % NSLOT

        @pl.when(b >= 2)
        def _():
          scatter_wait(j2)

        @pl.when(b + 2 < nbb)
        def _():
          gather_start(b + 2, j2)

      return carry

    lax.fori_loop(0, nbb // NSLOT, outer, 0)
    scatter_wait((nbb - 2) % NSLOT)
    scatter_wait((nbb - 1) % NSLOT)
    plsc.subcore_barrier()
    pltpu.sync_copy(acc.at[pl.ds(rbase, RPT)],
                    out_hbm.at[cid, pl.ds(rbase, RPT)])

  return k


_scatter_deg = _edge_scatter(16, sub=4)
_scatter64 = _edge_scatter(64, sub=2)
_scatter32 = _edge_scatter(32, sub=4)


R_BLK = 400
GRID = N // R_BLK


def _row_spec(c):
  return pl.BlockSpec((R_BLK, c), lambda i: (i, 0))


def _full_spec(r, c):
  return pl.BlockSpec((r, c), lambda i: (0, 0))


def _part_spec(c):
  return pl.BlockSpec((2, R_BLK, c), lambda i: (0, i, 0))


def _tc1(data, w1r, tw, deg2):
  def body(d_ref, w_ref, tw_ref, dg_ref, y0_ref, sl1_ref, dinv_ref):
    xw = jnp.dot(d_ref[...], w_ref[...],
                 preferred_element_type=jnp.float32) + tw_ref[...]
    deg = dg_ref[0, :, 0:1] + dg_ref[1, :, 0:1] + 1.0
    dinv = jnp.where(deg > 0, lax.rsqrt(deg), 0.0)
    y0_ref[...] = dinv * xw
    sl1_ref[...] = (dinv * dinv) * xw
    dinv_ref[...] = dinv

  return pl.pallas_call(
      body,
      grid=(GRID,),
      in_specs=[_row_spec(128), _full_spec(128, 64), _full_spec(1, 64),
                _part_spec(16)],
      out_specs=[_row_spec(64), _row_spec(64), _row_spec(1)],
      out_shape=[
          jax.ShapeDtypeStruct((N, 64), jnp.float32),
          jax.ShapeDtypeStruct((N, 64), jnp.float32),
          jax.ShapeDtypeStruct((N, 1), jnp.float32),
      ],
  )(data, w1r, tw, deg2)


def _tc2(s1, sl1, dinv, b1, w2):
  def body(s_ref, sl_ref, dv_ref, b_ref, w_ref, y_ref, sl2_ref):
    dinv = dv_ref[...]
    h = jnp.tanh(dinv * (s_ref[0] + s_ref[1]) + sl_ref[...] + b_ref[...])
    xw = jnp.dot(h, w_ref[...], preferred_element_type=jnp.float32)
    y_ref[...] = dinv * xw
    sl2_ref[...] = (dinv * dinv) * xw

  return pl.pallas_call(
      body,
      grid=(GRID,),
      in_specs=[_part_spec(64), _row_spec(64), _row_spec(1),
                _full_spec(1, 64), _full_spec(64, 32)],
      out_specs=[_row_spec(32), _row_spec(32)],
      out_shape=[
          jax.ShapeDtypeStruct((N, 32), jnp.float32),
          jax.ShapeDtypeStruct((N, 32), jnp.float32),
      ],
  )(s1, sl1, dinv, b1, w2)


def _tc3(s2, sl2, dinv, b2):
  def body(s_ref, sl_ref, dv_ref, b_ref, y_ref, sl3_ref):
    dinv = dv_ref[...]
    h = jnp.tanh(dinv * (s_ref[0] + s_ref[1]) + sl_ref[...] + b_ref[...])
    y_ref[...] = dinv * h
    sl3_ref[...] = (dinv * dinv) * h

  return pl.pallas_call(
      body,
      grid=(GRID,),
      in_specs=[_part_spec(32), _row_spec(32), _row_spec(1),
                _full_spec(1, 32)],
      out_specs=[_row_spec(32), _row_spec(32)],
      out_shape=[
          jax.ShapeDtypeStruct((N, 32), jnp.float32),
          jax.ShapeDtypeStruct((N, 32), jnp.float32),
      ],
  )(s2, sl2, dinv, b2)


def _tc4(s3, sl3, dinv, w3, b3):
  def body(s_ref, sl_ref, dv_ref, w_ref, b_ref, o_ref):
    agg = dv_ref[...] * (s_ref[0] + s_ref[1]) + sl_ref[...]
    o_ref[...] = jnp.dot(agg, w_ref[...],
                         preferred_element_type=jnp.float32) + b_ref[...]

  return pl.pallas_call(
      body,
      grid=(GRID,),
      in_specs=[_part_spec(32), _row_spec(32), _row_spec(1),
                _full_spec(32, 128), _full_spec(1, 128)],
      out_specs=_row_spec(128),
      out_shape=jax.ShapeDtypeStruct((N, 128), jnp.float32),
  )(s3, sl3, dinv, w3, b3)


def kernel(t, data, edges, pos, edge_attr, W1, b1, W2, b2, W3, b3):
  del pos
  edges = edges.astype(jnp.int32)
  pad = jnp.zeros((2, EPAD - E), jnp.int32)
  edges = jnp.concatenate([edges, pad], axis=1)
  row, col = edges[0], edges[1]
  ew = jnp.concatenate(
      [edge_attr.astype(jnp.float32), jnp.zeros((EPAD - E,), jnp.float32)])
  data = data.astype(jnp.float32)

  ones16 = jnp.ones((N, 16), jnp.float32)
  z16 = jnp.zeros((NP, 16), jnp.float32)
  z64 = jnp.zeros((NP, 64), jnp.float32)
  z32 = jnp.zeros((NP, 32), jnp.float32)
  tw = (t * W1[0])[None, :]
  w1r = W1[1:]

  deg2 = _scatter_deg(ones16, row, col, ew, z16)[:, :N]
  y0, sl1, dinv = _tc1(data, w1r, tw, deg2)
  s1 = _scatter64(y0, row, col, ew, z64)[:, :N]
  y1, sl2 = _tc2(s1, sl1, dinv, b1[None, :], W2)
  s2 = _scatter32(y1, row, col, ew, z32)[:, :N]
  y2, sl3 = _tc3(s2, sl2, dinv, b2[None, :])
  s3 = _scatter32(y2, row, col, ew, z32)[:, :N]
  return _tc4(s3, sl3, dinv, W3, b3[None, :])


# speedup vs baseline: 1.1349x; 1.1349x over previous
import functools

import jax
import jax.numpy as jnp
from jax import lax
from jax.experimental import pallas as pl
from jax.experimental.pallas import tpu as pltpu
from jax.experimental.pallas import tpu_sc as plsc

N = 10000
NP = 10240
E = 320000
NC, NS = 2, 16
NW = NC * NS
B = 128
NB = 80
EPW = B * NB
EPAD = EPW * NW
NSLOT = 4
RPT = NP // NS


def _edge_scatter(C, sub, frac0=0.5, stage_table=True):
  mesh = plsc.VectorSubcoreMesh(core_axis_name="c", subcore_axis_name="s")
  bb = B * sub
  tot = 2 * (EPW // bb)
  nbb0 = int(round(tot * frac0 / NSLOT)) * NSLOT
  nbb1 = tot - nbb0
  assert nbb1 % NSLOT == 0
  epw0, epw1 = bb * nbb0, bb * nbb1

  scratch = [
      pltpu.VMEM((max(epw0, epw1),), jnp.int32),
      pltpu.VMEM((max(epw0, epw1),), jnp.float32),
      pltpu.VMEM_SHARED((NP, C), jnp.float32),
  ]
  if stage_table:
    scratch.append(pltpu.VMEM_SHARED((NP, C), jnp.float32))
  for _ in range(NSLOT):
    scratch.append(pltpu.VMEM((sub, B), jnp.int32))
  for _ in range(NSLOT):
    scratch.append(pltpu.VMEM((sub, B, C), jnp.float32))
  scratch += [pltpu.SemaphoreType.DMA] * (2 * NSLOT)

  @functools.partial(
      pl.kernel,
      out_type=jax.ShapeDtypeStruct((NC, NP, C), jnp.float32),
      mesh=mesh,
      scratch_types=scratch,
      compiler_params=pltpu.CompilerParams(use_tc_tiling_on_sc=False),
  )
  def k(y_hbm, row_hbm, col_hbm, ew_hbm, z_hbm, out_hbm, row_v, ew_v, acc,
        *bufs):
    if stage_table:
      tab, bufs = bufs[0], bufs[1:]
    else:
      tab = y_hbm
    colb = bufs[0:NSLOT]
    msg = bufs[NSLOT:2 * NSLOT]
    gsem = bufs[2 * NSLOT:3 * NSLOT]
    ssem = bufs[3 * NSLOT:4 * NSLOT]
    cid = lax.axis_index("c")
    sid = lax.axis_index("s")
    nbb = jnp.where(cid == 0, nbb0, nbb1)
    ebase = pl.multiple_of(
        jnp.where(cid == 0, sid * epw0, NS * epw0 + sid * epw1), 8)
    epwmax = max(epw0, epw1)
    pltpu.sync_copy(row_hbm.at[pl.ds(ebase, epwmax)], row_v)
    pltpu.sync_copy(ew_hbm.at[pl.ds(ebase, epwmax)], ew_v)
    rbase = pl.multiple_of(sid * RPT, 8)
    pltpu.sync_copy(z_hbm.at[pl.ds(rbase, RPT)], acc.at[pl.ds(rbase, RPT)])
    if stage_table:
      pltpu.sync_copy(y_hbm.at[pl.ds(rbase, RPT)], tab.at[pl.ds(rbase, RPT)])
    plsc.subcore_barrier()

    def gather_start(b, j):
      off = pl.multiple_of(b * bb, 8)
      for si in range(sub):
        pltpu.async_copy(col_hbm.at[pl.ds(ebase + off + si * B, B)],
                         colb[j].at[si], gsem[j])
        pltpu.async_copy(tab.at[row_v.at[pl.ds(off + si * B, B)]],
                         msg[j].at[si], gsem[j])

    def gather_wait(j):
      for si in range(sub):
        pltpu.make_async_copy(col_hbm.at[pl.ds(0, B)], colb[j].at[si],
                              gsem[j]).wait()
        pltpu.make_async_copy(tab.at[row_v.at[pl.ds(0, B)]], msg[j].at[si],
                              gsem[j]).wait()

    def scatter_start(j):
      for si in range(sub):
        pltpu.async_copy(msg[j].at[si], acc.at[colb[j].at[si]], ssem[j],
                         add=True)

    def scatter_wait(j):
      for si in range(sub):
        pltpu.make_async_copy(msg[j].at[si], acc.at[colb[j].at[si]],
                              ssem[j]).wait()

    def scale(b, j):
      off = pl.multiple_of(b * bb, 8)
      for si in range(sub):

        def grp(g, c2):
          ew16 = ew_v[pl.ds(off + si * B + g * 16, 16)]
          for jj in range(16):
            e = g * 16 + jj
            s = ew16.at[jnp.full((16,), jj, jnp.int32)].get(
                mode="promise_in_bounds")
            for cc in range(C // 16):
              msg[j][si, e, pl.ds(cc * 16, 16)] = (
                  msg[j][si, e, pl.ds(cc * 16, 16)] * s)
          return c2

        lax.fori_loop(0, B // 16, grp, 0)

    gather_start(0, 0)
    gather_start(1, 1)

    def outer(i, carry):
      for jj in range(NSLOT):
        b = i * NSLOT + jj
        gather_wait(jj)
        scale(b, jj)
        scatter_start(jj)
        j2 = (jj + 2) % NSLOT

        @pl.when(b >= 2)
        def _():
          scatter_wait(j2)

        @pl.when(b + 2 < nbb)
        def _():
          gather_start(b + 2, j2)

      return carry

    lax.fori_loop(0, nbb // NSLOT, outer, 0)
    scatter_wait(NSLOT - 2)
    scatter_wait(NSLOT - 1)
    plsc.subcore_barrier()
    pltpu.sync_copy(acc.at[pl.ds(rbase, RPT)],
                    out_hbm.at[cid, pl.ds(rbase, RPT)])

  return k


_scatter_deg = _edge_scatter(16, sub=4, frac0=0.6, stage_table=False)
_scatter64 = _edge_scatter(64, sub=2, frac0=0.55, stage_table=False)
_scatter32 = _edge_scatter(32, sub=4, frac0=0.6, stage_table=False)


R_BLK = 400
GRID = N // R_BLK


def _row_spec(c):
  return pl.BlockSpec((R_BLK, c), lambda i: (i, 0))


def _full_spec(r, c):
  return pl.BlockSpec((r, c), lambda i: (0, 0))


def _part_spec(c):
  return pl.BlockSpec((2, R_BLK, c), lambda i: (0, i, 0))


def _tc1(data, w1r, tw, deg2):
  def body(d_ref, w_ref, tw_ref, dg_ref, y0_ref, sl1_ref, dinv_ref):
    xw = jnp.dot(d_ref[...], w_ref[...],
                 preferred_element_type=jnp.float32) + tw_ref[...]
    deg = dg_ref[0, :, 0:1] + dg_ref[1, :, 0:1] + 1.0
    dinv = jnp.where(deg > 0, lax.rsqrt(deg), 0.0)
    y0_ref[...] = dinv * xw
    sl1_ref[...] = (dinv * dinv) * xw
    dinv_ref[...] = dinv

  return pl.pallas_call(
      body,
      grid=(GRID,),
      in_specs=[_row_spec(128), _full_spec(128, 64), _full_spec(1, 64),
                _part_spec(16)],
      out_specs=[_row_spec(64), _row_spec(64), _row_spec(1)],
      out_shape=[
          jax.ShapeDtypeStruct((NP, 64), jnp.float32),
          jax.ShapeDtypeStruct((N, 64), jnp.float32),
          jax.ShapeDtypeStruct((N, 1), jnp.float32),
      ],
  )(data, w1r, tw, deg2)


def _tc2(s1, sl1, dinv, b1, w2):
  def body(s_ref, sl_ref, dv_ref, b_ref, w_ref, y_ref, sl2_ref):
    dinv = dv_ref[...]
    h = jnp.tanh(dinv * (s_ref[0] + s_ref[1]) + sl_ref[...] + b_ref[...])
    xw = jnp.dot(h, w_ref[...], preferred_element_type=jnp.float32)
    y_ref[...] = dinv * xw
    sl2_ref[...] = (dinv * dinv) * xw

  return pl.pallas_call(
      body,
      grid=(GRID,),
      in_specs=[_part_spec(64), _row_spec(64), _row_spec(1),
                _full_spec(1, 64), _full_spec(64, 32)],
      out_specs=[_row_spec(32), _row_spec(32)],
      out_shape=[
          jax.ShapeDtypeStruct((NP, 32), jnp.float32),
          jax.ShapeDtypeStruct((N, 32), jnp.float32),
      ],
  )(s1, sl1, dinv, b1, w2)


def _tc3(s2, sl2, dinv, b2):
  def body(s_ref, sl_ref, dv_ref, b_ref, y_ref, sl3_ref):
    dinv = dv_ref[...]
    h = jnp.tanh(dinv * (s_ref[0] + s_ref[1]) + sl_ref[...] + b_ref[...])
    y_ref[...] = dinv * h
    sl3_ref[...] = (dinv * dinv) * h

  return pl.pallas_call(
      body,
      grid=(GRID,),
      in_specs=[_part_spec(32), _row_spec(32), _row_spec(1),
                _full_spec(1, 32)],
      out_specs=[_row_spec(32), _row_spec(32)],
      out_shape=[
          jax.ShapeDtypeStruct((NP, 32), jnp.float32),
          jax.ShapeDtypeStruct((N, 32), jnp.float32),
      ],
  )(s2, sl2, dinv, b2)


def _tc4(s3, sl3, dinv, w3, b3):
  def body(s_ref, sl_ref, dv_ref, w_ref, b_ref, o_ref):
    agg = dv_ref[...] * (s_ref[0] + s_ref[1]) + sl_ref[...]
    o_ref[...] = jnp.dot(agg, w_ref[...],
                         preferred_element_type=jnp.float32) + b_ref[...]

  return pl.pallas_call(
      body,
      grid=(GRID,),
      in_specs=[_part_spec(32), _row_spec(32), _row_spec(1),
                _full_spec(32, 128), _full_spec(1, 128)],
      out_specs=_row_spec(128),
      out_shape=jax.ShapeDtypeStruct((N, 128), jnp.float32),
  )(s3, sl3, dinv, w3, b3)


def kernel(t, data, edges, pos, edge_attr, W1, b1, W2, b2, W3, b3):
  del pos
  edges = edges.astype(jnp.int32)
  pad = jnp.zeros((2, EPAD + 8192 - E), jnp.int32)
  edges = jnp.concatenate([edges, pad], axis=1)
  row, col = edges[0], edges[1]
  ew = jnp.concatenate(
      [edge_attr.astype(jnp.float32),
       jnp.zeros((EPAD + 8192 - E,), jnp.float32)])
  data = data.astype(jnp.float32)

  ones16 = jnp.ones((NP, 16), jnp.float32)
  z16 = jnp.zeros((NP, 16), jnp.float32)
  z64 = jnp.zeros((NP, 64), jnp.float32)
  z32 = jnp.zeros((NP, 32), jnp.float32)
  tw = (t * W1[0])[None, :]
  w1r = W1[1:]

  deg2 = _scatter_deg(ones16, row, col, ew, z16)[:, :N]
  y0, sl1, dinv = _tc1(data, w1r, tw, deg2)
  s1 = _scatter64(y0, row, col, ew, z64)[:, :N]
  y1, sl2 = _tc2(s1, sl1, dinv, b1[None, :], W2)
  s2 = _scatter32(y1, row, col, ew, z32)[:, :N]
  y2, sl3 = _tc3(s2, sl2, dinv, b2[None, :])
  s3 = _scatter32(y2, row, col, ew, z32)[:, :N]
  return _tc4(s3, sl3, dinv, W3, b3[None, :])


# speedup vs baseline: 1.1632x; 1.0249x over previous
import functools

import jax
import jax.numpy as jnp
from jax import lax
from jax.experimental import pallas as pl
from jax.experimental.pallas import tpu as pltpu
from jax.experimental.pallas import tpu_sc as plsc

N = 10000
NP = 10240
E = 320000
NC, NS = 2, 16
NW = NC * NS
B = 128
NB = 80
EPW = B * NB
EPAD = EPW * NW
NSLOT = 4
RPT = NP // NS


def _edge_scatter(C, sub, frac0=0.5, stage_table=True):
  mesh = plsc.VectorSubcoreMesh(core_axis_name="c", subcore_axis_name="s")
  bb = B * sub
  tot = 2 * (EPW // bb)
  nbb0 = int(round(tot * frac0 / NSLOT)) * NSLOT
  nbb1 = tot - nbb0
  assert nbb1 % NSLOT == 0
  epw0, epw1 = bb * nbb0, bb * nbb1

  scratch = [
      pltpu.VMEM((max(epw0, epw1),), jnp.int32),
      pltpu.VMEM((max(epw0, epw1),), jnp.float32),
      pltpu.VMEM_SHARED((NP, C), jnp.float32),
  ]
  if stage_table:
    scratch.append(pltpu.VMEM_SHARED((NP, C), jnp.float32))
  for _ in range(NSLOT):
    scratch.append(pltpu.VMEM((sub, B), jnp.int32))
  for _ in range(NSLOT):
    scratch.append(pltpu.VMEM((sub, B, C), jnp.float32))
  scratch += [pltpu.SemaphoreType.DMA] * (2 * NSLOT)

  @functools.partial(
      pl.kernel,
      out_type=jax.ShapeDtypeStruct((NC, NP, C), jnp.float32),
      mesh=mesh,
      scratch_types=scratch,
      compiler_params=pltpu.CompilerParams(use_tc_tiling_on_sc=False),
  )
  def k(y_hbm, row_hbm, col_hbm, ew_hbm, z_hbm, out_hbm, row_v, ew_v, acc,
        *bufs):
    if stage_table:
      tab, bufs = bufs[0], bufs[1:]
    else:
      tab = y_hbm
    colb = bufs[0:NSLOT]
    msg = bufs[NSLOT:2 * NSLOT]
    gsem = bufs[2 * NSLOT:3 * NSLOT]
    ssem = bufs[3 * NSLOT:4 * NSLOT]
    cid = lax.axis_index("c")
    sid = lax.axis_index("s")
    nbb = jnp.where(cid == 0, nbb0, nbb1)
    ebase = pl.multiple_of(
        jnp.where(cid == 0, sid * epw0, NS * epw0 + sid * epw1), 8)
    epwmax = max(epw0, epw1)
    pltpu.sync_copy(row_hbm.at[pl.ds(ebase, epwmax)], row_v)
    pltpu.sync_copy(ew_hbm.at[pl.ds(ebase, epwmax)], ew_v)
    rbase = pl.multiple_of(sid * RPT, 8)
    pltpu.sync_copy(z_hbm.at[pl.ds(rbase, RPT)], acc.at[pl.ds(rbase, RPT)])
    if stage_table:
      pltpu.sync_copy(y_hbm.at[pl.ds(rbase, RPT)], tab.at[pl.ds(rbase, RPT)])
    plsc.subcore_barrier()

    def gather_start(b, j):
      off = pl.multiple_of(b * bb, 8)
      for si in range(sub):
        pltpu.async_copy(col_hbm.at[pl.ds(ebase + off + si * B, B)],
                         colb[j].at[si], gsem[j])
        pltpu.async_copy(tab.at[row_v.at[pl.ds(off + si * B, B)]],
                         msg[j].at[si], gsem[j])

    def gather_wait(j):
      for si in range(sub):
        pltpu.make_async_copy(col_hbm.at[pl.ds(0, B)], colb[j].at[si],
                              gsem[j]).wait()
        pltpu.make_async_copy(tab.at[row_v.at[pl.ds(0, B)]], msg[j].at[si],
                              gsem[j]).wait()

    def scatter_start(j):
      for si in range(sub):
        pltpu.async_copy(msg[j].at[si], acc.at[colb[j].at[si]], ssem[j],
                         add=True)

    def scatter_wait(j):
      for si in range(sub):
        pltpu.make_async_copy(msg[j].at[si], acc.at[colb[j].at[si]],
                              ssem[j]).wait()

    def scale(b, j):
      off = pl.multiple_of(b * bb, 8)
      for si in range(sub):

        def grp(g, c2):
          ew16 = ew_v[pl.ds(off + si * B + g * 16, 16)]
          for jj in range(16):
            e = g * 16 + jj
            s = ew16.at[jnp.full((16,), jj, jnp.int32)].get(
                mode="promise_in_bounds")
            for cc in range(C // 16):
              msg[j][si, e, pl.ds(cc * 16, 16)] = (
                  msg[j][si, e, pl.ds(cc * 16, 16)] * s)
          return c2

        lax.fori_loop(0, B // 16, grp, 0)

    gather_start(0, 0)
    gather_start(1, 1)

    def outer(i, carry):
      for jj in range(NSLOT):
        b = i * NSLOT + jj
        gather_wait(jj)
        scale(b, jj)
        scatter_start(jj)
        j2 = (jj + 2) % NSLOT

        @pl.when(b >= 2)
        def _():
          scatter_wait(j2)

        @pl.when(b + 2 < nbb)
        def _():
          gather_start(b + 2, j2)

      return carry

    lax.fori_loop(0, nbb // NSLOT, outer, 0)
    scatter_wait(NSLOT - 2)
    scatter_wait(NSLOT - 1)
    plsc.subcore_barrier()
    pltpu.sync_copy(acc.at[pl.ds(rbase, RPT)],
                    out_hbm.at[cid, pl.ds(rbase, RPT)])

  return k


_scatter_deg = _edge_scatter(16, sub=4, frac0=0.6, stage_table=False)
_scatter64 = _edge_scatter(64, sub=1, frac0=0.6, stage_table=False)
_scatter32 = _edge_scatter(32, sub=4, frac0=0.6, stage_table=False)


R_BLK = 400
GRID = N // R_BLK


def _row_spec(c):
  return pl.BlockSpec((R_BLK, c), lambda i: (i, 0))


def _full_spec(r, c):
  return pl.BlockSpec((r, c), lambda i: (0, 0))


def _part_spec(c):
  return pl.BlockSpec((2, R_BLK, c), lambda i: (0, i, 0))


def _tc1(data, w1r, tw, deg2):
  def body(d_ref, w_ref, tw_ref, dg_ref, y0_ref, sl1_ref, dinv_ref):
    xw = jnp.dot(d_ref[...], w_ref[...],
                 preferred_element_type=jnp.float32) + tw_ref[...]
    deg = dg_ref[0, :, 0:1] + dg_ref[1, :, 0:1] + 1.0
    dinv = jnp.where(deg > 0, lax.rsqrt(deg), 0.0)
    y0_ref[...] = dinv * xw
    sl1_ref[...] = (dinv * dinv) * xw
    dinv_ref[...] = dinv

  return pl.pallas_call(
      body,
      grid=(GRID,),
      in_specs=[_row_spec(128), _full_spec(128, 64), _full_spec(1, 64),
                _part_spec(16)],
      out_specs=[_row_spec(64), _row_spec(64), _row_spec(1)],
      out_shape=[
          jax.ShapeDtypeStruct((NP, 64), jnp.float32),
          jax.ShapeDtypeStruct((N, 64), jnp.float32),
          jax.ShapeDtypeStruct((N, 1), jnp.float32),
      ],
  )(data, w1r, tw, deg2)


def _tc2(s1, sl1, dinv, b1, w2):
  def body(s_ref, sl_ref, dv_ref, b_ref, w_ref, y_ref, sl2_ref):
    dinv = dv_ref[...]
    h = jnp.tanh(dinv * (s_ref[0] + s_ref[1]) + sl_ref[...] + b_ref[...])
    xw = jnp.dot(h, w_ref[...], preferred_element_type=jnp.float32)
    y_ref[...] = dinv * xw
    sl2_ref[...] = (dinv * dinv) * xw

  return pl.pallas_call(
      body,
      grid=(GRID,),
      in_specs=[_part_spec(64), _row_spec(64), _row_spec(1),
                _full_spec(1, 64), _full_spec(64, 32)],
      out_specs=[_row_spec(32), _row_spec(32)],
      out_shape=[
          jax.ShapeDtypeStruct((NP, 32), jnp.float32),
          jax.ShapeDtypeStruct((N, 32), jnp.float32),
      ],
  )(s1, sl1, dinv, b1, w2)


def _tc3(s2, sl2, dinv, b2):
  def body(s_ref, sl_ref, dv_ref, b_ref, y_ref, sl3_ref):
    dinv = dv_ref[...]
    h = jnp.tanh(dinv * (s_ref[0] + s_ref[1]) + sl_ref[...] + b_ref[...])
    y_ref[...] = dinv * h
    sl3_ref[...] = (dinv * dinv) * h

  return pl.pallas_call(
      body,
      grid=(GRID,),
      in_specs=[_part_spec(32), _row_spec(32), _row_spec(1),
                _full_spec(1, 32)],
      out_specs=[_row_spec(32), _row_spec(32)],
      out_shape=[
          jax.ShapeDtypeStruct((NP, 32), jnp.float32),
          jax.ShapeDtypeStruct((N, 32), jnp.float32),
      ],
  )(s2, sl2, dinv, b2)


def _tc4(s3, sl3, dinv, w3, b3):
  def body(s_ref, sl_ref, dv_ref, w_ref, b_ref, o_ref):
    agg = dv_ref[...] * (s_ref[0] + s_ref[1]) + sl_ref[...]
    o_ref[...] = jnp.dot(agg, w_ref[...],
                         preferred_element_type=jnp.float32) + b_ref[...]

  return pl.pallas_call(
      body,
      grid=(GRID,),
      in_specs=[_part_spec(32), _row_spec(32), _row_spec(1),
                _full_spec(32, 128), _full_spec(1, 128)],
      out_specs=_row_spec(128),
      out_shape=jax.ShapeDtypeStruct((N, 128), jnp.float32),
  )(s3, sl3, dinv, w3, b3)


def kernel(t, data, edges, pos, edge_attr, W1, b1, W2, b2, W3, b3):
  del pos
  edges = edges.astype(jnp.int32)
  pad = jnp.zeros((2, EPAD + 8192 - E), jnp.int32)
  edges = jnp.concatenate([edges, pad], axis=1)
  row, col = edges[0], edges[1]
  ew = jnp.concatenate(
      [edge_attr.astype(jnp.float32),
       jnp.zeros((EPAD + 8192 - E,), jnp.float32)])
  data = data.astype(jnp.float32)

  ones16 = jnp.ones((NP, 16), jnp.float32)
  z16 = jnp.zeros((NP, 16), jnp.float32)
  z64 = jnp.zeros((NP, 64), jnp.float32)
  z32 = jnp.zeros((NP, 32), jnp.float32)
  tw = (t * W1[0])[None, :]
  w1r = W1[1:]

  deg2 = _scatter_deg(ones16, row, col, ew, z16)[:, :N]
  y0, sl1, dinv = _tc1(data, w1r, tw, deg2)
  s1 = _scatter64(y0, row, col, ew, z64)[:, :N]
  y1, sl2 = _tc2(s1, sl1, dinv, b1[None, :], W2)
  s2 = _scatter32(y1, row, col, ew, z32)[:, :N]
  y2, sl3 = _tc3(s2, sl2, dinv, b2[None, :])
  s3 = _scatter32(y2, row, col, ew, z32)[:, :N]
  return _tc4(s3, sl3, dinv, W3, b3[None, :])


# speedup vs baseline: 1.1852x; 1.0189x over previous
import functools

import jax
import jax.numpy as jnp
from jax import lax
from jax.experimental import pallas as pl
from jax.experimental.pallas import tpu as pltpu
from jax.experimental.pallas import tpu_sc as plsc

N = 10000
NP = 10240
E = 320000
NC, NS = 2, 16
NW = NC * NS
B = 128
NB = 80
EPW = B * NB
EPAD = EPW * NW
NSLOT = 4
RPT = NP // NS


def _edge_scatter(C, sub, frac0=0.5, stage_table=True, deg_mode=False):
  mesh = plsc.VectorSubcoreMesh(core_axis_name="c", subcore_axis_name="s")
  bb = B * sub
  tot = 2 * (EPW // bb)
  nbb0 = int(round(tot * frac0 / NSLOT)) * NSLOT
  nbb1 = tot - nbb0
  assert nbb1 % NSLOT == 0
  epw0, epw1 = bb * nbb0, bb * nbb1

  scratch = [
      pltpu.VMEM((max(epw0, epw1),), jnp.int32),
      pltpu.VMEM((max(epw0, epw1),), jnp.float32),
      pltpu.VMEM_SHARED((NP, C), jnp.float32),
  ]
  if stage_table:
    scratch.append(pltpu.VMEM_SHARED((NP, C), jnp.float32))
  for _ in range(NSLOT):
    scratch.append(pltpu.VMEM((sub, B), jnp.int32))
  for _ in range(NSLOT):
    scratch.append(pltpu.VMEM((sub, B, C), jnp.float32))
  scratch += [pltpu.SemaphoreType.DMA] * (2 * NSLOT)

  @functools.partial(
      pl.kernel,
      out_type=jax.ShapeDtypeStruct((NC, NP, C), jnp.float32),
      mesh=mesh,
      scratch_types=scratch,
      compiler_params=pltpu.CompilerParams(use_tc_tiling_on_sc=False),
  )
  def k(y_hbm, row_hbm, col_hbm, ew_hbm, z_hbm, out_hbm, row_v, ew_v, acc,
        *bufs):
    if stage_table:
      tab, bufs = bufs[0], bufs[1:]
    else:
      tab = y_hbm
    colb = bufs[0:NSLOT]
    msg = bufs[NSLOT:2 * NSLOT]
    gsem = bufs[2 * NSLOT:3 * NSLOT]
    ssem = bufs[3 * NSLOT:4 * NSLOT]
    cid = lax.axis_index("c")
    sid = lax.axis_index("s")
    nbb = jnp.where(cid == 0, nbb0, nbb1)
    ebase = pl.multiple_of(
        jnp.where(cid == 0, sid * epw0, NS * epw0 + sid * epw1), 8)
    epwmax = max(epw0, epw1)
    pltpu.sync_copy(row_hbm.at[pl.ds(ebase, epwmax)], row_v)
    pltpu.sync_copy(ew_hbm.at[pl.ds(ebase, epwmax)], ew_v)
    rbase = pl.multiple_of(sid * RPT, 8)
    pltpu.sync_copy(z_hbm.at[pl.ds(rbase, RPT)], acc.at[pl.ds(rbase, RPT)])
    if deg_mode:
      for j in range(NSLOT):
        for si in range(sub):
          pltpu.sync_copy(z_hbm.at[pl.ds(0, B)], msg[j].at[si])
    if stage_table:
      pltpu.sync_copy(y_hbm.at[pl.ds(rbase, RPT)], tab.at[pl.ds(rbase, RPT)])
    plsc.subcore_barrier()

    def gather_start(b, j):
      off = pl.multiple_of(b * bb, 8)
      for si in range(sub):
        pltpu.async_copy(col_hbm.at[pl.ds(ebase + off + si * B, B)],
                         colb[j].at[si], gsem[j])
        if not deg_mode:
          pltpu.async_copy(tab.at[row_v.at[pl.ds(off + si * B, B)]],
                           msg[j].at[si], gsem[j])

    def gather_wait(j):
      for si in range(sub):
        pltpu.make_async_copy(col_hbm.at[pl.ds(0, B)], colb[j].at[si],
                              gsem[j]).wait()
        if not deg_mode:
          pltpu.make_async_copy(tab.at[row_v.at[pl.ds(0, B)]], msg[j].at[si],
                                gsem[j]).wait()

    def scatter_start(j):
      for si in range(sub):
        pltpu.async_copy(msg[j].at[si], acc.at[colb[j].at[si]], ssem[j],
                         add=True)

    def scatter_wait(j):
      for si in range(sub):
        pltpu.make_async_copy(msg[j].at[si], acc.at[colb[j].at[si]],
                              ssem[j]).wait()

    def scale(b, j):
      off = pl.multiple_of(b * bb, 8)
      for si in range(sub):

        def grp(g, c2):
          ew16 = ew_v[pl.ds(off + si * B + g * 16, 16)]
          for jj in range(16):
            e = g * 16 + jj
            s = ew16.at[jnp.full((16,), jj, jnp.int32)].get(
                mode="promise_in_bounds")
            if deg_mode:
              msg[j][si, e, pl.ds(0, 16)] = (
                  msg[j][si, e, pl.ds(0, 16)] * 0.0 + s)
            else:
              for cc in range(C // 16):
                msg[j][si, e, pl.ds(cc * 16, 16)] = (
                    msg[j][si, e, pl.ds(cc * 16, 16)] * s)
          return c2

        lax.fori_loop(0, B // 16, grp, 0)

    gather_start(0, 0)
    gather_start(1, 1)

    def outer(i, carry):
      for jj in range(NSLOT):
        b = i * NSLOT + jj
        gather_wait(jj)
        scale(b, jj)
        scatter_start(jj)
        j2 = (jj + 2) % NSLOT

        @pl.when(b >= 2)
        def _():
          scatter_wait(j2)

        @pl.when(b + 2 < nbb)
        def _():
          gather_start(b + 2, j2)

      return carry

    lax.fori_loop(0, nbb // NSLOT, outer, 0)
    scatter_wait(NSLOT - 2)
    scatter_wait(NSLOT - 1)
    plsc.subcore_barrier()
    pltpu.sync_copy(acc.at[pl.ds(rbase, RPT)],
                    out_hbm.at[cid, pl.ds(rbase, RPT)])

  return k


_scatter_deg = _edge_scatter(16, sub=4, frac0=0.6, stage_table=False, deg_mode=True)
_scatter64 = _edge_scatter(64, sub=1, frac0=0.6, stage_table=False)
_scatter32 = _edge_scatter(32, sub=4, frac0=0.6, stage_table=False)


R_BLK = 400
GRID = N // R_BLK


def _row_spec(c):
  return pl.BlockSpec((R_BLK, c), lambda i: (i, 0))


def _full_spec(r, c):
  return pl.BlockSpec((r, c), lambda i: (0, 0))


def _part_spec(c):
  return pl.BlockSpec((2, R_BLK, c), lambda i: (0, i, 0))


def _tc1(data, w1r, tw, deg2):
  def body(d_ref, w_ref, tw_ref, dg_ref, y0_ref, sl1_ref, dinv_ref):
    xw = jnp.dot(d_ref[...], w_ref[...],
                 preferred_element_type=jnp.float32) + tw_ref[...]
    deg = dg_ref[0, :, 0:1] + dg_ref[1, :, 0:1] + 1.0
    dinv = jnp.where(deg > 0, lax.rsqrt(deg), 0.0)
    y0_ref[...] = dinv * xw
    sl1_ref[...] = (dinv * dinv) * xw
    dinv_ref[...] = dinv

  return pl.pallas_call(
      body,
      grid=(GRID,),
      in_specs=[_row_spec(128), _full_spec(128, 64), _full_spec(1, 64),
                _part_spec(16)],
      out_specs=[_row_spec(64), _row_spec(64), _row_spec(1)],
      out_shape=[
          jax.ShapeDtypeStruct((NP, 64), jnp.float32),
          jax.ShapeDtypeStruct((N, 64), jnp.float32),
          jax.ShapeDtypeStruct((N, 1), jnp.float32),
      ],
  )(data, w1r, tw, deg2)


def _tc2(s1, sl1, dinv, b1, w2):
  def body(s_ref, sl_ref, dv_ref, b_ref, w_ref, y_ref, sl2_ref):
    dinv = dv_ref[...]
    h = jnp.tanh(dinv * (s_ref[0] + s_ref[1]) + sl_ref[...] + b_ref[...])
    xw = jnp.dot(h, w_ref[...], preferred_element_type=jnp.float32)
    y_ref[...] = dinv * xw
    sl2_ref[...] = (dinv * dinv) * xw

  return pl.pallas_call(
      body,
      grid=(GRID,),
      in_specs=[_part_spec(64), _row_spec(64), _row_spec(1),
                _full_spec(1, 64), _full_spec(64, 32)],
      out_specs=[_row_spec(32), _row_spec(32)],
      out_shape=[
          jax.ShapeDtypeStruct((NP, 32), jnp.float32),
          jax.ShapeDtypeStruct((N, 32), jnp.float32),
      ],
  )(s1, sl1, dinv, b1, w2)


def _tc3(s2, sl2, dinv, b2):
  def body(s_ref, sl_ref, dv_ref, b_ref, y_ref, sl3_ref):
    dinv = dv_ref[...]
    h = jnp.tanh(dinv * (s_ref[0] + s_ref[1]) + sl_ref[...] + b_ref[...])
    y_ref[...] = dinv * h
    sl3_ref[...] = (dinv * dinv) * h

  return pl.pallas_call(
      body,
      grid=(GRID,),
      in_specs=[_part_spec(32), _row_spec(32), _row_spec(1),
                _full_spec(1, 32)],
      out_specs=[_row_spec(32), _row_spec(32)],
      out_shape=[
          jax.ShapeDtypeStruct((NP, 32), jnp.float32),
          jax.ShapeDtypeStruct((N, 32), jnp.float32),
      ],
  )(s2, sl2, dinv, b2)


def _tc4(s3, sl3, dinv, w3, b3):
  def body(s_ref, sl_ref, dv_ref, w_ref, b_ref, o_ref):
    agg = dv_ref[...] * (s_ref[0] + s_ref[1]) + sl_ref[...]
    o_ref[...] = jnp.dot(agg, w_ref[...],
                         preferred_element_type=jnp.float32) + b_ref[...]

  return pl.pallas_call(
      body,
      grid=(GRID,),
      in_specs=[_part_spec(32), _row_spec(32), _row_spec(1),
                _full_spec(32, 128), _full_spec(1, 128)],
      out_specs=_row_spec(128),
      out_shape=jax.ShapeDtypeStruct((N, 128), jnp.float32),
  )(s3, sl3, dinv, w3, b3)


def kernel(t, data, edges, pos, edge_attr, W1, b1, W2, b2, W3, b3):
  del pos
  edges = edges.astype(jnp.int32)
  pad = jnp.zeros((2, EPAD + 8192 - E), jnp.int32)
  edges = jnp.concatenate([edges, pad], axis=1)
  row, col = edges[0], edges[1]
  ew = jnp.concatenate(
      [edge_attr.astype(jnp.float32),
       jnp.zeros((EPAD + 8192 - E,), jnp.float32)])
  data = data.astype(jnp.float32)

  ones16 = jnp.ones((NP, 16), jnp.float32)
  z16 = jnp.zeros((NP, 16), jnp.float32)
  z64 = jnp.zeros((NP, 64), jnp.float32)
  z32 = jnp.zeros((NP, 32), jnp.float32)
  tw = (t * W1[0])[None, :]
  w1r = W1[1:]

  deg2 = _scatter_deg(ones16, row, col, ew, z16)[:, :N]
  y0, sl1, dinv = _tc1(data, w1r, tw, deg2)
  s1 = _scatter64(y0, row, col, ew, z64)[:, :N]
  y1, sl2 = _tc2(s1, sl1, dinv, b1[None, :], W2)
  s2 = _scatter32(y1, row, col, ew, z32)[:, :N]
  y2, sl3 = _tc3(s2, sl2, dinv, b2[None, :])
  s3 = _scatter32(y2, row, col, ew, z32)[:, :N]
  return _tc4(s3, sl3, dinv, W3, b3[None, :])


# speedup vs baseline: 1.3877x; 1.1709x over previous
import functools

import jax
import jax.numpy as jnp
from jax import lax
from jax.experimental import pallas as pl
from jax.experimental.pallas import tpu as pltpu
from jax.experimental.pallas import tpu_sc as plsc

N = 10000
NP = 10240
E = 320000
NC, NS = 2, 16
NW = NC * NS
B = 128
NB = 80
EPW = B * NB
EPAD = EPW * NW
NSLOT = 4
RPT = NP // NS


def _edge_scatter(C, sub, frac0=0.5, stage_table=True, deg_mode=False,
                  bf16_table=False):
  mesh = plsc.VectorSubcoreMesh(core_axis_name="c", subcore_axis_name="s")
  bb = B * sub
  tot = 2 * (EPW // bb)
  nbb0 = int(round(tot * frac0 / NSLOT)) * NSLOT
  nbb1 = tot - nbb0
  assert nbb1 % NSLOT == 0
  epw0, epw1 = bb * nbb0, bb * nbb1

  scratch = [
      pltpu.VMEM((max(epw0, epw1),), jnp.int32),
      pltpu.VMEM((max(epw0, epw1),), jnp.float32),
      pltpu.VMEM_SHARED((NP, C), jnp.float32),
  ]
  if stage_table:
    scratch.append(pltpu.VMEM_SHARED((NP, C), jnp.float32))
  for _ in range(NSLOT):
    scratch.append(pltpu.VMEM((sub, B), jnp.int32))
  for _ in range(NSLOT):
    scratch.append(pltpu.VMEM((sub, B, C), jnp.float32))
  if bf16_table:
    for _ in range(NSLOT):
      scratch.append(pltpu.VMEM((sub, B, C // 2), jnp.int32))
  scratch += [pltpu.SemaphoreType.DMA] * (2 * NSLOT)

  @functools.partial(
      pl.kernel,
      out_type=jax.ShapeDtypeStruct((NC, NP, C), jnp.float32),
      mesh=mesh,
      scratch_types=scratch,
      compiler_params=pltpu.CompilerParams(use_tc_tiling_on_sc=False),
  )
  def k(y_hbm, row_hbm, col_hbm, ew_hbm, z_hbm, out_hbm, row_v, ew_v, acc,
        *bufs):
    if stage_table:
      tab, bufs = bufs[0], bufs[1:]
    else:
      tab = y_hbm
    colb = bufs[0:NSLOT]
    msg = bufs[NSLOT:2 * NSLOT]
    bufs = bufs[2 * NSLOT:]
    if bf16_table:
      msgb = bufs[0:NSLOT]
      bufs = bufs[NSLOT:]
    else:
      msgb = msg
    gsem = bufs[0:NSLOT]
    ssem = bufs[NSLOT:2 * NSLOT]
    cid = lax.axis_index("c")
    sid = lax.axis_index("s")
    nbb = jnp.where(cid == 0, nbb0, nbb1)
    ebase = pl.multiple_of(
        jnp.where(cid == 0, sid * epw0, NS * epw0 + sid * epw1), 8)
    epwmax = max(epw0, epw1)
    pltpu.sync_copy(row_hbm.at[pl.ds(ebase, epwmax)], row_v)
    pltpu.sync_copy(ew_hbm.at[pl.ds(ebase, epwmax)], ew_v)
    rbase = pl.multiple_of(sid * RPT, 8)
    pltpu.sync_copy(z_hbm.at[pl.ds(rbase, RPT)], acc.at[pl.ds(rbase, RPT)])
    if deg_mode:
      for j in range(NSLOT):
        for si in range(sub):
          pltpu.sync_copy(z_hbm.at[pl.ds(0, B)], msg[j].at[si])
    if stage_table:
      pltpu.sync_copy(y_hbm.at[pl.ds(rbase, RPT)], tab.at[pl.ds(rbase, RPT)])
    plsc.subcore_barrier()

    def gather_start(b, j):
      off = pl.multiple_of(b * bb, 8)
      for si in range(sub):
        pltpu.async_copy(col_hbm.at[pl.ds(ebase + off + si * B, B)],
                         colb[j].at[si], gsem[j])
        if not deg_mode:
          pltpu.async_copy(tab.at[row_v.at[pl.ds(off + si * B, B)]],
                           msgb[j].at[si], gsem[j])

    def gather_wait(j):
      for si in range(sub):
        pltpu.make_async_copy(col_hbm.at[pl.ds(0, B)], colb[j].at[si],
                              gsem[j]).wait()
        if not deg_mode:
          pltpu.make_async_copy(tab.at[row_v.at[pl.ds(0, B)]], msgb[j].at[si],
                                gsem[j]).wait()

    def scatter_start(j):
      for si in range(sub):
        pltpu.async_copy(msg[j].at[si], acc.at[colb[j].at[si]], ssem[j],
                         add=True)

    def scatter_wait(j):
      for si in range(sub):
        pltpu.make_async_copy(msg[j].at[si], acc.at[colb[j].at[si]],
                              ssem[j]).wait()

    def scale(b, j):
      off = pl.multiple_of(b * bb, 8)
      for si in range(sub):

        def grp(g, c2):
          ew16 = ew_v[pl.ds(off + si * B + g * 16, 16)]
          for jj in range(16):
            e = g * 16 + jj
            s = ew16.at[jnp.full((16,), jj, jnp.int32)].get(
                mode="promise_in_bounds")
            if deg_mode:
              msg[j][si, e, pl.ds(0, 16)] = (
                  msg[j][si, e, pl.ds(0, 16)] * 0.0 + s)
            elif bf16_table:
              for g32 in range(C // 32):
                v = msgb[j][si, e, pl.ds(g32 * 16, 16)]
                lo = lax.bitcast_convert_type(v << 16, jnp.float32)
                hi = lax.bitcast_convert_type(v & jnp.int32(-65536),
                                              jnp.float32)
                msg[j][si, e, pl.ds(g32 * 32, 16)] = lo * s
                msg[j][si, e, pl.ds(g32 * 32 + 16, 16)] = hi * s
            else:
              for cc in range(C // 16):
                msg[j][si, e, pl.ds(cc * 16, 16)] = (
                    msg[j][si, e, pl.ds(cc * 16, 16)] * s)
          return c2

        lax.fori_loop(0, B // 16, grp, 0)

    gather_start(0, 0)
    gather_start(1, 1)

    def outer(i, carry):
      for jj in range(NSLOT):
        b = i * NSLOT + jj
        gather_wait(jj)
        scale(b, jj)
        scatter_start(jj)
        j2 = (jj + 2) % NSLOT

        @pl.when(b >= 2)
        def _():
          scatter_wait(j2)

        @pl.when(b + 2 < nbb)
        def _():
          gather_start(b + 2, j2)

      return carry

    lax.fori_loop(0, nbb // NSLOT, outer, 0)
    scatter_wait(NSLOT - 2)
    scatter_wait(NSLOT - 1)
    plsc.subcore_barrier()
    pltpu.sync_copy(acc.at[pl.ds(rbase, RPT)],
                    out_hbm.at[cid, pl.ds(rbase, RPT)])

  return k


_scatter_deg = _edge_scatter(16, sub=4, frac0=0.6, stage_table=False, deg_mode=True)
_scatter64 = _edge_scatter(64, sub=1, frac0=0.6, stage_table=False,
                           bf16_table=True)
_scatter32 = _edge_scatter(32, sub=2, frac0=0.6, stage_table=False,
                           bf16_table=True)


R_BLK = 400
GRID = N // R_BLK


def _row_spec(c):
  return pl.BlockSpec((R_BLK, c), lambda i: (i, 0))


def _full_spec(r, c):
  return pl.BlockSpec((r, c), lambda i: (0, 0))


def _part_spec(c):
  return pl.BlockSpec((2, R_BLK, c), lambda i: (0, i, 0))


def _tc1(data, w1r, tw, deg2):
  def body(d_ref, w_ref, tw_ref, dg_ref, y0_ref, sl1_ref, dinv_ref):
    xw = jnp.dot(d_ref[...], w_ref[...],
                 preferred_element_type=jnp.float32) + tw_ref[...]
    deg = dg_ref[0, :, 0:1] + dg_ref[1, :, 0:1] + 1.0
    dinv = jnp.where(deg > 0, lax.rsqrt(deg), 0.0)
    y0_ref[...] = dinv * xw
    sl1_ref[...] = (dinv * dinv) * xw
    dinv_ref[...] = dinv

  return pl.pallas_call(
      body,
      grid=(GRID,),
      in_specs=[_row_spec(128), _full_spec(128, 64), _full_spec(1, 64),
                _part_spec(16)],
      out_specs=[_row_spec(64), _row_spec(64), _row_spec(1)],
      out_shape=[
          jax.ShapeDtypeStruct((NP, 64), jnp.float32),
          jax.ShapeDtypeStruct((N, 64), jnp.float32),
          jax.ShapeDtypeStruct((N, 1), jnp.float32),
      ],
  )(data, w1r, tw, deg2)


def _tc2(s1, sl1, dinv, b1, w2):
  def body(s_ref, sl_ref, dv_ref, b_ref, w_ref, y_ref, sl2_ref):
    dinv = dv_ref[...]
    h = jnp.tanh(dinv * (s_ref[0] + s_ref[1]) + sl_ref[...] + b_ref[...])
    xw = jnp.dot(h, w_ref[...], preferred_element_type=jnp.float32)
    y_ref[...] = dinv * xw
    sl2_ref[...] = (dinv * dinv) * xw

  return pl.pallas_call(
      body,
      grid=(GRID,),
      in_specs=[_part_spec(64), _row_spec(64), _row_spec(1),
                _full_spec(1, 64), _full_spec(64, 32)],
      out_specs=[_row_spec(32), _row_spec(32)],
      out_shape=[
          jax.ShapeDtypeStruct((NP, 32), jnp.float32),
          jax.ShapeDtypeStruct((N, 32), jnp.float32),
      ],
  )(s1, sl1, dinv, b1, w2)


def _tc3(s2, sl2, dinv, b2):
  def body(s_ref, sl_ref, dv_ref, b_ref, y_ref, sl3_ref):
    dinv = dv_ref[...]
    h = jnp.tanh(dinv * (s_ref[0] + s_ref[1]) + sl_ref[...] + b_ref[...])
    y_ref[...] = dinv * h
    sl3_ref[...] = (dinv * dinv) * h

  return pl.pallas_call(
      body,
      grid=(GRID,),
      in_specs=[_part_spec(32), _row_spec(32), _row_spec(1),
                _full_spec(1, 32)],
      out_specs=[_row_spec(32), _row_spec(32)],
      out_shape=[
          jax.ShapeDtypeStruct((NP, 32), jnp.float32),
          jax.ShapeDtypeStruct((N, 32), jnp.float32),
      ],
  )(s2, sl2, dinv, b2)


def _tc4(s3, sl3, dinv, w3, b3):
  def body(s_ref, sl_ref, dv_ref, w_ref, b_ref, o_ref):
    agg = dv_ref[...] * (s_ref[0] + s_ref[1]) + sl_ref[...]
    o_ref[...] = jnp.dot(agg, w_ref[...],
                         preferred_element_type=jnp.float32) + b_ref[...]

  return pl.pallas_call(
      body,
      grid=(GRID,),
      in_specs=[_part_spec(32), _row_spec(32), _row_spec(1),
                _full_spec(32, 128), _full_spec(1, 128)],
      out_specs=_row_spec(128),
      out_shape=jax.ShapeDtypeStruct((N, 128), jnp.float32),
  )(s3, sl3, dinv, w3, b3)


def _ileave32(y):
  r, c = y.shape
  y4 = y.reshape(r, c // 32, 2, 16)
  y4 = jnp.swapaxes(y4, 2, 3)
  yb = y4.reshape(r, c // 2, 2).astype(jnp.bfloat16)
  return jax.lax.bitcast_convert_type(yb, jnp.int32)


def kernel(t, data, edges, pos, edge_attr, W1, b1, W2, b2, W3, b3):
  del pos
  edges = edges.astype(jnp.int32)
  pad = jnp.zeros((2, EPAD + 8192 - E), jnp.int32)
  edges = jnp.concatenate([edges, pad], axis=1)
  row, col = edges[0], edges[1]
  ew = jnp.concatenate(
      [edge_attr.astype(jnp.float32),
       jnp.zeros((EPAD + 8192 - E,), jnp.float32)])
  data = data.astype(jnp.float32)

  ones16 = jnp.ones((NP, 16), jnp.float32)
  z16 = jnp.zeros((NP, 16), jnp.float32)
  z64 = jnp.zeros((NP, 64), jnp.float32)
  z32 = jnp.zeros((NP, 32), jnp.float32)
  tw = (t * W1[0])[None, :]
  w1r = W1[1:]

  deg2 = _scatter_deg(ones16, row, col, ew, z16)[:, :N]
  y0, sl1, dinv = _tc1(data, w1r, tw, deg2)
  s1 = _scatter64(_ileave32(y0), row, col, ew, z64)[:, :N]
  y1, sl2 = _tc2(s1, sl1, dinv, b1[None, :], W2)
  s2 = _scatter32(_ileave32(y1), row, col, ew, z32)[:, :N]
  y2, sl3 = _tc3(s2, sl2, dinv, b2[None, :])
  s3 = _scatter32(_ileave32(y2), row, col, ew, z32)[:, :N]
  return _tc4(s3, sl3, dinv, W3, b3[None, :])


# speedup vs baseline: 1.3887x; 1.0007x over previous
import functools

import jax
import jax.numpy as jnp
from jax import lax
from jax.experimental import pallas as pl
from jax.experimental.pallas import tpu as pltpu
from jax.experimental.pallas import tpu_sc as plsc

N = 10000
NP = 10240
E = 320000
NC, NS = 2, 16
NW = NC * NS
B = 128
NB = 80
EPW = B * NB
EPAD = EPW * NW
NSLOT = 4
RPT = NP // NS


def _edge_scatter(C, sub, frac0=0.5, deg_mode=False, bf16_table=False):
  mesh = plsc.VectorSubcoreMesh(core_axis_name="c", subcore_axis_name="s")
  bb = B * sub
  tot = 2 * (EPW // bb)
  nbb0 = int(round(tot * frac0 / NSLOT)) * NSLOT
  nbb1 = tot - nbb0
  assert nbb1 % NSLOT == 0
  epw0, epw1 = bb * nbb0, bb * nbb1

  scratch = [
      pltpu.VMEM((max(epw0, epw1),), jnp.int32),
      pltpu.VMEM((max(epw0, epw1),), jnp.float32),
      pltpu.VMEM_SHARED((NP, C), jnp.float32),
  ]
  for _ in range(NSLOT):
    scratch.append(pltpu.VMEM((sub, B), jnp.int32))
  for _ in range(NSLOT):
    scratch.append(pltpu.VMEM((sub, B, C), jnp.float32))
  if bf16_table:
    for _ in range(NSLOT):
      scratch.append(pltpu.VMEM((sub, B, C // 2), jnp.int32))
  scratch += [pltpu.SemaphoreType.DMA] * (2 * NSLOT)

  @functools.partial(
      pl.kernel,
      out_type=jax.ShapeDtypeStruct((NC, NP, C), jnp.float32),
      mesh=mesh,
      scratch_types=scratch,
      compiler_params=pltpu.CompilerParams(use_tc_tiling_on_sc=False),
  )
  def k(y_hbm, row_hbm, col_hbm, ew_hbm, z_hbm, out_hbm, row_v, ew_v, acc,
        *bufs):
    tab = y_hbm
    colb = bufs[0:NSLOT]
    msg = bufs[NSLOT:2 * NSLOT]
    bufs = bufs[2 * NSLOT:]
    if bf16_table:
      msgb = bufs[0:NSLOT]
      bufs = bufs[NSLOT:]
    else:
      msgb = msg
    gsem = bufs[0:NSLOT]
    ssem = bufs[NSLOT:2 * NSLOT]
    cid = lax.axis_index("c")
    sid = lax.axis_index("s")
    nbb = jnp.where(cid == 0, nbb0, nbb1)
    ebase = pl.multiple_of(
        jnp.where(cid == 0, sid * epw0, NS * epw0 + sid * epw1), 8)
    epwmax = max(epw0, epw1)
    pltpu.sync_copy(row_hbm.at[pl.ds(ebase, epwmax)], row_v)
    pltpu.sync_copy(ew_hbm.at[pl.ds(ebase, epwmax)], ew_v)
    rbase = pl.multiple_of(sid * RPT, 8)
    pltpu.sync_copy(z_hbm.at[pl.ds(rbase, RPT)], acc.at[pl.ds(rbase, RPT)])
    if deg_mode:
      for j in range(NSLOT):
        for si in range(sub):
          pltpu.sync_copy(z_hbm.at[pl.ds(0, B)], msg[j].at[si])
    plsc.subcore_barrier()

    def gather_start(b, j):
      off = pl.multiple_of(b * bb, 8)
      for si in range(sub):
        pltpu.async_copy(col_hbm.at[pl.ds(ebase + off + si * B, B)],
                         colb[j].at[si], gsem[j])
        if not deg_mode:
          pltpu.async_copy(tab.at[row_v.at[pl.ds(off + si * B, B)]],
                           msgb[j].at[si], gsem[j])

    def gather_wait(j):
      for si in range(sub):
        pltpu.make_async_copy(col_hbm.at[pl.ds(0, B)], colb[j].at[si],
                              gsem[j]).wait()
        if not deg_mode:
          pltpu.make_async_copy(tab.at[row_v.at[pl.ds(0, B)]], msgb[j].at[si],
                                gsem[j]).wait()

    def scatter_start(j):
      for si in range(sub):
        pltpu.async_copy(msg[j].at[si], acc.at[colb[j].at[si]], ssem[j],
                         add=True)

    def scatter_wait(j):
      for si in range(sub):
        pltpu.make_async_copy(msg[j].at[si], acc.at[colb[j].at[si]],
                              ssem[j]).wait()

    def scale(b, j):
      off = pl.multiple_of(b * bb, 8)
      for si in range(sub):

        def grp(g, c2):
          ew16 = ew_v[pl.ds(off + si * B + g * 16, 16)]
          for jj in range(16):
            e = g * 16 + jj
            s = ew16.at[jnp.full((16,), jj, jnp.int32)].get(
                mode="promise_in_bounds")
            if deg_mode:
              msg[j][si, e, pl.ds(0, 16)] = (
                  msg[j][si, e, pl.ds(0, 16)] * 0.0 + s)
            elif bf16_table:
              for g32 in range(C // 32):
                v = msgb[j][si, e, pl.ds(g32 * 16, 16)]
                lo = lax.bitcast_convert_type(v << 16, jnp.float32)
                hi = lax.bitcast_convert_type(v & jnp.int32(-65536),
                                              jnp.float32)
                msg[j][si, e, pl.ds(g32 * 32, 16)] = lo * s
                msg[j][si, e, pl.ds(g32 * 32 + 16, 16)] = hi * s
            else:
              for cc in range(C // 16):
                msg[j][si, e, pl.ds(cc * 16, 16)] = (
                    msg[j][si, e, pl.ds(cc * 16, 16)] * s)
          return c2

        lax.fori_loop(0, B // 16, grp, 0)

    gather_start(0, 0)
    gather_start(1, 1)

    def outer(i, carry):
      for jj in range(NSLOT):
        b = i * NSLOT + jj
        gather_wait(jj)
        scale(b, jj)
        scatter_start(jj)
        j2 = (jj + 2) % NSLOT

        @pl.when(b >= 2)
        def _():
          scatter_wait(j2)

        @pl.when(b + 2 < nbb)
        def _():
          gather_start(b + 2, j2)

      return carry

    lax.fori_loop(0, nbb // NSLOT, outer, 0)
    scatter_wait(NSLOT - 2)
    scatter_wait(NSLOT - 1)
    plsc.subcore_barrier()
    pltpu.sync_copy(acc.at[pl.ds(rbase, RPT)],
                    out_hbm.at[cid, pl.ds(rbase, RPT)])

  return k


_scatter_deg = _edge_scatter(16, sub=4, frac0=0.6, deg_mode=True)
_scatter64 = _edge_scatter(64, sub=1, frac0=0.6, bf16_table=True)
_scatter32 = _edge_scatter(32, sub=2, frac0=0.6, bf16_table=True)


R_BLK = 400
GRID = N // R_BLK


def _row_spec(c):
  return pl.BlockSpec((R_BLK, c), lambda i: (i, 0))


def _full_spec(r, c):
  return pl.BlockSpec((r, c), lambda i: (0, 0))


def _part_spec(c):
  return pl.BlockSpec((2, R_BLK, c), lambda i: (0, i, 0))


def _tc1(data, w1r, tw, deg2):
  def body(d_ref, w_ref, tw_ref, dg_ref, y0_ref, sl1_ref, dinv_ref):
    xw = jnp.dot(d_ref[...], w_ref[...],
                 preferred_element_type=jnp.float32) + tw_ref[...]
    deg = dg_ref[0, :, 0:1] + dg_ref[1, :, 0:1] + 1.0
    dinv = jnp.where(deg > 0, lax.rsqrt(deg), 0.0)
    y0_ref[...] = dinv * xw
    sl1_ref[...] = (dinv * dinv) * xw
    dinv_ref[...] = dinv

  return pl.pallas_call(
      body,
      grid=(GRID,),
      in_specs=[_row_spec(128), _full_spec(128, 64), _full_spec(1, 64),
                _part_spec(16)],
      out_specs=[_row_spec(64), _row_spec(64), _row_spec(1)],
      out_shape=[
          jax.ShapeDtypeStruct((NP, 64), jnp.float32),
          jax.ShapeDtypeStruct((N, 64), jnp.float32),
          jax.ShapeDtypeStruct((N, 1), jnp.float32),
      ],
  )(data, w1r, tw, deg2)


def _tc2(s1, sl1, dinv, b1, w2):
  def body(s_ref, sl_ref, dv_ref, b_ref, w_ref, y_ref, sl2_ref):
    dinv = dv_ref[...]
    h = jnp.tanh(dinv * (s_ref[0] + s_ref[1]) + sl_ref[...] + b_ref[...])
    xw = jnp.dot(h, w_ref[...], preferred_element_type=jnp.float32)
    y_ref[...] = dinv * xw
    sl2_ref[...] = (dinv * dinv) * xw

  return pl.pallas_call(
      body,
      grid=(GRID,),
      in_specs=[_part_spec(64), _row_spec(64), _row_spec(1),
                _full_spec(1, 64), _full_spec(64, 32)],
      out_specs=[_row_spec(32), _row_spec(32)],
      out_shape=[
          jax.ShapeDtypeStruct((NP, 32), jnp.float32),
          jax.ShapeDtypeStruct((N, 32), jnp.float32),
      ],
  )(s1, sl1, dinv, b1, w2)


def _tc3(s2, sl2, dinv, b2):
  def body(s_ref, sl_ref, dv_ref, b_ref, y_ref, sl3_ref):
    dinv = dv_ref[...]
    h = jnp.tanh(dinv * (s_ref[0] + s_ref[1]) + sl_ref[...] + b_ref[...])
    y_ref[...] = dinv * h
    sl3_ref[...] = (dinv * dinv) * h

  return pl.pallas_call(
      body,
      grid=(GRID,),
      in_specs=[_part_spec(32), _row_spec(32), _row_spec(1),
                _full_spec(1, 32)],
      out_specs=[_row_spec(32), _row_spec(32)],
      out_shape=[
          jax.ShapeDtypeStruct((NP, 32), jnp.float32),
          jax.ShapeDtypeStruct((N, 32), jnp.float32),
      ],
  )(s2, sl2, dinv, b2)


def _tc4(s3, sl3, dinv, w3, b3):
  def body(s_ref, sl_ref, dv_ref, w_ref, b_ref, o_ref):
    agg = dv_ref[...] * (s_ref[0] + s_ref[1]) + sl_ref[...]
    o_ref[...] = jnp.dot(agg, w_ref[...],
                         preferred_element_type=jnp.float32) + b_ref[...]

  return pl.pallas_call(
      body,
      grid=(GRID,),
      in_specs=[_part_spec(32), _row_spec(32), _row_spec(1),
                _full_spec(32, 128), _full_spec(1, 128)],
      out_specs=_row_spec(128),
      out_shape=jax.ShapeDtypeStruct((N, 128), jnp.float32),
  )(s3, sl3, dinv, w3, b3)


def _ileave32(y):
  r, c = y.shape
  y4 = y.reshape(r, c // 32, 2, 16)
  y4 = jnp.swapaxes(y4, 2, 3)
  yb = y4.reshape(r, c // 2, 2).astype(jnp.bfloat16)
  return jax.lax.bitcast_convert_type(yb, jnp.int32)


def kernel(t, data, edges, pos, edge_attr, W1, b1, W2, b2, W3, b3):
  del pos
  edges = edges.astype(jnp.int32)
  pad = jnp.zeros((2, EPAD + 8192 - E), jnp.int32)
  edges = jnp.concatenate([edges, pad], axis=1)
  row, col = edges[0], edges[1]
  ew = jnp.concatenate(
      [edge_attr.astype(jnp.float32),
       jnp.zeros((EPAD + 8192 - E,), jnp.float32)])
  data = data.astype(jnp.float32)

  ones16 = jnp.ones((NP, 16), jnp.float32)
  z16 = jnp.zeros((NP, 16), jnp.float32)
  z64 = jnp.zeros((NP, 64), jnp.float32)
  z32 = jnp.zeros((NP, 32), jnp.float32)
  tw = (t * W1[0])[None, :]
  w1r = W1[1:]

  deg2 = _scatter_deg(ones16, row, col, ew, z16)[:, :N]
  y0, sl1, dinv = _tc1(data, w1r, tw, deg2)
  s1 = _scatter64(_ileave32(y0), row, col, ew, z64)[:, :N]
  y1, sl2 = _tc2(s1, sl1, dinv, b1[None, :], W2)
  s2 = _scatter32(_ileave32(y1), row, col, ew, z32)[:, :N]
  y2, sl3 = _tc3(s2, sl2, dinv, b2[None, :])
  s3 = _scatter32(_ileave32(y2), row, col, ew, z32)[:, :N]
  return _tc4(s3, sl3, dinv, W3, b3[None, :])
